# Initial kernel scaffold; baseline (speedup 1.0000x reference)
import jax
import jax.numpy as jnp
from jax.experimental import pallas as pl


def kernel(x, src_coords, src_batch, tgt_coords, tgt_batch, W1, b1, W2, b2, W3, b3, W4, b4):
    raise NotImplementedError("write your pallas kernel here")


# speedup vs baseline: 2.1111x; 2.1111x over previous
import functools

import jax
import jax.numpy as jnp
from jax import lax
from jax.experimental import pallas as pl
from jax.experimental.pallas import tpu as pltpu
from jax.experimental.pallas import tpu_sc as plsc

IN_C = 128
OUT_C = 128
HID = 256
NX = 8
NY = 8
NB = 4
NSEG = NB * NX * NY
N = 50000
NPAD = 50176
BLK = 512
NGRID = NPAD // BLK
NC = 2
NS = 16
NW = NC * NS
RPW = NPAD // NW
SB = 112
NBLK = RPW // SB
UW = 384


def _src_pool_body(idx_ref, x_ref, w1_ref, b1_ref, sums_ref):
    i = pl.program_id(0)
    u = jnp.dot(x_ref[...], w1_ref[...], preferred_element_type=jnp.float32)
    u = jax.nn.gelu(u + b1_ref[...])
    u_ext = jnp.concatenate(
        [u, jnp.ones((BLK, UW - HID), jnp.float32)], axis=1)
    rid = i * BLK + jax.lax.broadcasted_iota(jnp.int32, (BLK, 1), 0)
    u_ext = jnp.where(rid < N, u_ext, 0.0)
    seg = idx_ref[...]
    oh = (seg == jax.lax.broadcasted_iota(jnp.int32, (BLK, NSEG), 1)
          ).astype(jnp.float32)
    contrib = jax.lax.dot_general(
        oh, u_ext, (((0,), (0,)), ((), ())),
        preferred_element_type=jnp.float32)

    @pl.when(i == 0)
    def _():
        sums_ref[...] = contrib

    @pl.when(i > 0)
    def _():
        sums_ref[...] += contrib


_src_pool = pl.pallas_call(
    _src_pool_body,
    grid=(NGRID,),
    in_specs=[
        pl.BlockSpec((BLK, 1), lambda i: (i, 0)),
        pl.BlockSpec((BLK, IN_C), lambda i: (i, 0)),
        pl.BlockSpec((IN_C, HID), lambda i: (0, 0)),
        pl.BlockSpec((1, HID), lambda i: (0, 0)),
    ],
    out_specs=pl.BlockSpec((NSEG, UW), lambda i: (0, 0)),
    out_shape=jax.ShapeDtypeStruct((NSEG, UW), jnp.float32),
    compiler_params=pltpu.CompilerParams(dimension_semantics=("arbitrary",)),
)


def _pool_proj_body(sums_ref, w2_ref, b2_ref, w3p_ref, tbl_ref):
    full = sums_ref[...]
    su = full[:, 0:HID]
    cnt0 = full[:, HID:HID + 1]
    m = jnp.dot(su, w2_ref[...], preferred_element_type=jnp.float32)
    m = m + cnt0 * b2_ref[...]
    pooled = m / jnp.maximum(cnt0, 1.0)
    tbl_ref[...] = jnp.dot(pooled, w3p_ref[...],
                           preferred_element_type=jnp.float32)


_pool_proj = pl.pallas_call(
    _pool_proj_body,
    out_shape=jax.ShapeDtypeStruct((NSEG, HID), jnp.float32),
)


@functools.lru_cache(maxsize=None)
def _make_sc_gather():
    mesh = plsc.VectorSubcoreMesh(core_axis_name="c", subcore_axis_name="s")

    @functools.partial(
        pl.kernel,
        mesh=mesh,
        out_type=jax.ShapeDtypeStruct((NPAD, HID), jnp.float32),
        scratch_types=[
            pltpu.VMEM((NBLK, SB), jnp.int32),
            pltpu.VMEM((SB, HID), jnp.float32),
            pltpu.SemaphoreType.DMA,
        ],
    )
    def _sc_gather(tbl_hbm, idx_hbm, g_out, idx_v, rows_v, sem):
        c = lax.axis_index("c")
        s = lax.axis_index("s")
        w = s * NC + c
        pltpu.sync_copy(idx_hbm.at[w], idx_v)
        base = w * RPW
        for j in range(NBLK):
            pltpu.async_copy(tbl_hbm.at[idx_v.at[j]], rows_v, sem).wait()
            pltpu.sync_copy(rows_v, g_out.at[pl.ds(base + j * SB, SB)])

    return _sc_gather


def _tgt_mlp_body(g_ref, c_ref, w3c_ref, b3_ref, w4_ref, b4_ref, o_ref):
    cc = c_ref[...]
    w3c = w3c_ref[...]
    h = g_ref[...] + cc[:, 0:1] * w3c[0:1, :] + cc[:, 1:2] * w3c[1:2, :]
    h = jax.nn.gelu(h + b3_ref[...])
    o_ref[...] = jnp.dot(h, w4_ref[...],
                         preferred_element_type=jnp.float32) + b4_ref[...]


_tgt_mlp = pl.pallas_call(
    _tgt_mlp_body,
    grid=(NGRID,),
    in_specs=[
        pl.BlockSpec((BLK, HID), lambda i: (i, 0)),
        pl.BlockSpec((BLK, 2), lambda i: (i, 0)),
        pl.BlockSpec((2, HID), lambda i: (0, 0)),
        pl.BlockSpec((1, HID), lambda i: (0, 0)),
        pl.BlockSpec((HID, OUT_C), lambda i: (0, 0)),
        pl.BlockSpec((1, OUT_C), lambda i: (0, 0)),
    ],
    out_specs=pl.BlockSpec((BLK, OUT_C), lambda i: (i, 0)),
    out_shape=jax.ShapeDtypeStruct((N, OUT_C), jnp.float32),
    compiler_params=pltpu.CompilerParams(dimension_semantics=("parallel",)),
)


def _clusters(coords, batch):
    cx = jnp.clip(jnp.floor(coords[:, 0] * NX).astype(jnp.int32), 0, NX - 1)
    cy = jnp.clip(jnp.floor(coords[:, 1] * NY).astype(jnp.int32), 0, NY - 1)
    return batch.astype(jnp.int32) * (NX * NY) + cx * NY + cy


def kernel(x, src_coords, src_batch, tgt_coords, tgt_batch,
           W1, b1, W2, b2, W3, b3, W4, b4):
    src_sub = _clusters(src_coords, src_batch)
    tgt_sub = _clusters(tgt_coords, tgt_batch)
    src_idx2 = jnp.zeros((NPAD, 1), jnp.int32).at[:N, 0].set(src_sub)
    tgt_idx = jnp.zeros((NPAD,), jnp.int32).at[:N].set(tgt_sub)
    tgt_idx3 = tgt_idx.reshape(NW, NBLK, SB)

    sums = _src_pool(src_idx2, x, W1, b1.reshape(1, HID))
    tbl = _pool_proj(sums, W2, b2.reshape(1, HID), W3[2:])
    g = _make_sc_gather()(tbl, tgt_idx3)
    out = _tgt_mlp(g, tgt_coords, W3[:2], b3.reshape(1, HID),
                   W4, b4.reshape(1, OUT_C))
    return out


# speedup vs baseline: 2.2224x; 1.0527x over previous
import functools

import jax
import jax.numpy as jnp
from jax import lax
from jax.experimental import pallas as pl
from jax.experimental.pallas import tpu as pltpu
from jax.experimental.pallas import tpu_sc as plsc

IN_C = 128
OUT_C = 128
HID = 256
NX = 8
NY = 8
NB = 4
NSEG = NB * NX * NY
N = 50000
NPAD = 50176
BLK = 512
NGRID = NPAD // BLK
NC = 2
NS = 16
NW = NC * NS
RPW = NPAD // NW
SB = 112
NBLK = RPW // SB
UW = 384


def _src_pool_body(idx_ref, x_ref, w1_ref, b1_ref, sums_ref):
    i = pl.program_id(0)
    u = jnp.dot(x_ref[...], w1_ref[...], preferred_element_type=jnp.float32)
    u = jax.nn.gelu(u + b1_ref[...])
    u_ext = jnp.concatenate(
        [u, jnp.ones((BLK, UW - HID), jnp.float32)], axis=1)
    rid = i * BLK + jax.lax.broadcasted_iota(jnp.int32, (BLK, 1), 0)
    u_ext = jnp.where(rid < N, u_ext, 0.0)
    seg = idx_ref[...]
    oh = (seg == jax.lax.broadcasted_iota(jnp.int32, (BLK, NSEG), 1)
          ).astype(jnp.float32)
    contrib = jax.lax.dot_general(
        oh, u_ext, (((0,), (0,)), ((), ())),
        preferred_element_type=jnp.float32)

    @pl.when(i == 0)
    def _():
        sums_ref[...] = contrib

    @pl.when(i > 0)
    def _():
        sums_ref[...] += contrib


_src_pool = pl.pallas_call(
    _src_pool_body,
    grid=(NGRID,),
    in_specs=[
        pl.BlockSpec((BLK, 1), lambda i: (i, 0)),
        pl.BlockSpec((BLK, IN_C), lambda i: (i, 0)),
        pl.BlockSpec((IN_C, HID), lambda i: (0, 0)),
        pl.BlockSpec((1, HID), lambda i: (0, 0)),
    ],
    out_specs=pl.BlockSpec((NSEG, UW), lambda i: (0, 0)),
    out_shape=jax.ShapeDtypeStruct((NSEG, UW), jnp.float32),
    compiler_params=pltpu.CompilerParams(dimension_semantics=("arbitrary",)),
)


def _pool_proj_body(sums_ref, w2_ref, b2_ref, w3p_ref, tbl_ref):
    full = sums_ref[...]
    su = full[:, 0:HID]
    cnt0 = full[:, HID:HID + 1]
    m = jnp.dot(su, w2_ref[...], preferred_element_type=jnp.float32)
    m = m + cnt0 * b2_ref[...]
    pooled = m / jnp.maximum(cnt0, 1.0)
    tbl_ref[...] = jnp.dot(pooled, w3p_ref[...],
                           preferred_element_type=jnp.float32)


NREP = NW

_pool_proj = pl.pallas_call(
    _pool_proj_body,
    grid=(NREP,),
    in_specs=[
        pl.BlockSpec((NSEG, UW), lambda r: (0, 0)),
        pl.BlockSpec((HID, HID), lambda r: (0, 0)),
        pl.BlockSpec((1, HID), lambda r: (0, 0)),
        pl.BlockSpec((HID, HID), lambda r: (0, 0)),
    ],
    out_specs=pl.BlockSpec((NSEG, HID), lambda r: (r, 0)),
    out_shape=jax.ShapeDtypeStruct((NREP * NSEG, HID), jnp.float32),
    compiler_params=pltpu.CompilerParams(dimension_semantics=("parallel",)),
)


@functools.lru_cache(maxsize=None)
def _make_sc_gather():
    mesh = plsc.VectorSubcoreMesh(core_axis_name="c", subcore_axis_name="s")

    @functools.partial(
        pl.kernel,
        mesh=mesh,
        out_type=jax.ShapeDtypeStruct((NPAD, HID), jnp.float32),
        scratch_types=[
            pltpu.VMEM((NBLK, SB), jnp.int32),
            pltpu.VMEM((SB, HID), jnp.float32),
            pltpu.SemaphoreType.DMA,
        ],
    )
    def _sc_gather(tbl_hbm, idx_hbm, g_out, idx_v, rows_v, sem):
        c = lax.axis_index("c")
        s = lax.axis_index("s")
        w = s * NC + c
        pltpu.sync_copy(idx_hbm.at[w], idx_v)
        base = w * RPW
        for j in range(NBLK):
            pltpu.async_copy(tbl_hbm.at[idx_v.at[j]], rows_v, sem).wait()
            pltpu.sync_copy(rows_v, g_out.at[pl.ds(base + j * SB, SB)])

    return _sc_gather


def _tgt_mlp_body(g_ref, c_ref, w3c_ref, b3_ref, w4_ref, b4_ref, o_ref):
    cc = c_ref[...]
    w3c = w3c_ref[...]
    h = g_ref[...] + cc[:, 0:1] * w3c[0:1, :] + cc[:, 1:2] * w3c[1:2, :]
    h = jax.nn.gelu(h + b3_ref[...])
    o_ref[...] = jnp.dot(h, w4_ref[...],
                         preferred_element_type=jnp.float32) + b4_ref[...]


_tgt_mlp = pl.pallas_call(
    _tgt_mlp_body,
    grid=(NGRID,),
    in_specs=[
        pl.BlockSpec((BLK, HID), lambda i: (i, 0)),
        pl.BlockSpec((BLK, 2), lambda i: (i, 0)),
        pl.BlockSpec((2, HID), lambda i: (0, 0)),
        pl.BlockSpec((1, HID), lambda i: (0, 0)),
        pl.BlockSpec((HID, OUT_C), lambda i: (0, 0)),
        pl.BlockSpec((1, OUT_C), lambda i: (0, 0)),
    ],
    out_specs=pl.BlockSpec((BLK, OUT_C), lambda i: (i, 0)),
    out_shape=jax.ShapeDtypeStruct((N, OUT_C), jnp.float32),
    compiler_params=pltpu.CompilerParams(dimension_semantics=("parallel",)),
)


def _clusters(coords, batch):
    cx = jnp.clip(jnp.floor(coords[:, 0] * NX).astype(jnp.int32), 0, NX - 1)
    cy = jnp.clip(jnp.floor(coords[:, 1] * NY).astype(jnp.int32), 0, NY - 1)
    return batch.astype(jnp.int32) * (NX * NY) + cx * NY + cy


def kernel(x, src_coords, src_batch, tgt_coords, tgt_batch,
           W1, b1, W2, b2, W3, b3, W4, b4):
    src_sub = _clusters(src_coords, src_batch)
    tgt_sub = _clusters(tgt_coords, tgt_batch)
    src_idx2 = jnp.zeros((NPAD, 1), jnp.int32).at[:N, 0].set(src_sub)
    tgt_idx = jnp.zeros((NPAD,), jnp.int32).at[:N].set(tgt_sub)
    tgt_idx3 = tgt_idx.reshape(NW, NBLK, SB)
    rep_of_w = (jnp.arange(NW, dtype=jnp.int32) % NREP) * NSEG
    tgt_idx3 = tgt_idx3 + rep_of_w[:, None, None]

    sums = _src_pool(src_idx2, x, W1, b1.reshape(1, HID))
    tbl = _pool_proj(sums, W2, b2.reshape(1, HID), W3[2:])
    g = _make_sc_gather()(tbl, tgt_idx3)
    out = _tgt_mlp(g, tgt_coords, W3[:2], b3.reshape(1, HID),
                   W4, b4.reshape(1, OUT_C))
    return out


# speedup vs baseline: 2.4847x; 1.1180x over previous
import functools

import jax
import jax.numpy as jnp
from jax import lax
from jax.experimental import pallas as pl
from jax.experimental.pallas import tpu as pltpu
from jax.experimental.pallas import tpu_sc as plsc

IN_C = 128
OUT_C = 128
HID = 256
NX = 8
NY = 8
NB = 4
NSEG = NB * NX * NY
N = 50000
NPAD = 50176
BLK = 512
NGRID = NPAD // BLK
NC = 2
NS = 16
NW = NC * NS
RPW = NPAD // NW
SB = 112
NBLK = RPW // SB
UW = 384


def _src_pool_body(idx_ref, x_ref, w1_ref, b1_ref, sums_ref):
    i = pl.program_id(0)
    u = jnp.dot(x_ref[...], w1_ref[...], preferred_element_type=jnp.float32)
    u = jax.nn.gelu(u + b1_ref[...])
    u_ext = jnp.concatenate(
        [u, jnp.ones((BLK, UW - HID), jnp.float32)], axis=1)
    rid = i * BLK + jax.lax.broadcasted_iota(jnp.int32, (BLK, 1), 0)
    u_ext = jnp.where(rid < N, u_ext, 0.0)
    seg = idx_ref[0]
    oht = (seg == jax.lax.broadcasted_iota(jnp.int32, (NSEG, BLK), 0)
           ).astype(jnp.float32)
    contrib = jnp.dot(oht, u_ext,
                      preferred_element_type=jnp.float32)

    @pl.when(i == 0)
    def _():
        sums_ref[...] = contrib

    @pl.when(i > 0)
    def _():
        sums_ref[...] += contrib


_src_pool = pl.pallas_call(
    _src_pool_body,
    grid=(NGRID,),
    in_specs=[
        pl.BlockSpec((1, 1, BLK), lambda i: (i, 0, 0)),
        pl.BlockSpec((BLK, IN_C), lambda i: (i, 0)),
        pl.BlockSpec((IN_C, HID), lambda i: (0, 0)),
        pl.BlockSpec((1, HID), lambda i: (0, 0)),
    ],
    out_specs=pl.BlockSpec((NSEG, UW), lambda i: (0, 0)),
    out_shape=jax.ShapeDtypeStruct((NSEG, UW), jnp.float32),
    compiler_params=pltpu.CompilerParams(dimension_semantics=("arbitrary",)),
)


def _pool_proj_body(sums_ref, w2_ref, b2_ref, w3p_ref, tbl_ref):
    full = sums_ref[...]
    su = full[:, 0:HID]
    cnt0 = full[:, HID:HID + 1]
    m = jnp.dot(su, w2_ref[...], preferred_element_type=jnp.float32)
    m = m + cnt0 * b2_ref[...]
    pooled = m / jnp.maximum(cnt0, 1.0)
    tbl_ref[...] = jnp.dot(pooled, w3p_ref[...],
                           preferred_element_type=jnp.float32)


NREP = NW

_pool_proj = pl.pallas_call(
    _pool_proj_body,
    grid=(NREP,),
    in_specs=[
        pl.BlockSpec((NSEG, UW), lambda r: (0, 0)),
        pl.BlockSpec((HID, HID), lambda r: (0, 0)),
        pl.BlockSpec((1, HID), lambda r: (0, 0)),
        pl.BlockSpec((HID, HID), lambda r: (0, 0)),
    ],
    out_specs=pl.BlockSpec((NSEG, HID), lambda r: (r, 0)),
    out_shape=jax.ShapeDtypeStruct((NREP * NSEG, HID), jnp.float32),
    compiler_params=pltpu.CompilerParams(dimension_semantics=("parallel",)),
)


@functools.lru_cache(maxsize=None)
def _make_sc_gather():
    mesh = plsc.VectorSubcoreMesh(core_axis_name="c", subcore_axis_name="s")

    @functools.partial(
        pl.kernel,
        mesh=mesh,
        out_type=jax.ShapeDtypeStruct((NPAD, HID), jnp.float32),
        scratch_types=[
            pltpu.VMEM((NBLK, SB), jnp.int32),
            pltpu.VMEM((SB, HID), jnp.float32),
            pltpu.SemaphoreType.DMA,
        ],
    )
    def _sc_gather(tbl_hbm, idx_hbm, g_out, idx_v, rows_v, sem):
        c = lax.axis_index("c")
        s = lax.axis_index("s")
        w = s * NC + c
        pltpu.sync_copy(idx_hbm.at[w], idx_v)
        base = w * RPW
        for j in range(NBLK):
            pltpu.async_copy(tbl_hbm.at[idx_v.at[j]], rows_v, sem).wait()
            pltpu.sync_copy(rows_v, g_out.at[pl.ds(base + j * SB, SB)])

    return _sc_gather


def _tgt_mlp_body(g_ref, c_ref, w3c_ref, b3_ref, w4_ref, b4_ref, o_ref):
    cc = c_ref[...]
    w3c = w3c_ref[...]
    h = g_ref[...] + cc[:, 0:1] * w3c[0:1, :] + cc[:, 1:2] * w3c[1:2, :]
    h = jax.nn.gelu(h + b3_ref[...])
    o_ref[...] = jnp.dot(h, w4_ref[...],
                         preferred_element_type=jnp.float32) + b4_ref[...]


_tgt_mlp = pl.pallas_call(
    _tgt_mlp_body,
    grid=(NGRID,),
    in_specs=[
        pl.BlockSpec((BLK, HID), lambda i: (i, 0)),
        pl.BlockSpec((BLK, 2), lambda i: (i, 0)),
        pl.BlockSpec((2, HID), lambda i: (0, 0)),
        pl.BlockSpec((1, HID), lambda i: (0, 0)),
        pl.BlockSpec((HID, OUT_C), lambda i: (0, 0)),
        pl.BlockSpec((1, OUT_C), lambda i: (0, 0)),
    ],
    out_specs=pl.BlockSpec((BLK, OUT_C), lambda i: (i, 0)),
    out_shape=jax.ShapeDtypeStruct((N, OUT_C), jnp.float32),
    compiler_params=pltpu.CompilerParams(dimension_semantics=("parallel",)),
)


def _clusters(coords, batch):
    cx = jnp.clip(jnp.floor(coords[:, 0] * NX).astype(jnp.int32), 0, NX - 1)
    cy = jnp.clip(jnp.floor(coords[:, 1] * NY).astype(jnp.int32), 0, NY - 1)
    return batch.astype(jnp.int32) * (NX * NY) + cx * NY + cy


def kernel(x, src_coords, src_batch, tgt_coords, tgt_batch,
           W1, b1, W2, b2, W3, b3, W4, b4):
    src_sub = _clusters(src_coords, src_batch)
    tgt_sub = _clusters(tgt_coords, tgt_batch)
    src_idx3d = jnp.full((NPAD,), NSEG, jnp.int32).at[:N].set(src_sub)
    src_idx3d = src_idx3d.reshape(NGRID, 1, BLK)
    tgt_idx = jnp.zeros((NPAD,), jnp.int32).at[:N].set(tgt_sub)
    tgt_idx3 = tgt_idx.reshape(NW, NBLK, SB)
    rep_of_w = (jnp.arange(NW, dtype=jnp.int32) % NREP) * NSEG
    tgt_idx3 = tgt_idx3 + rep_of_w[:, None, None]

    sums = _src_pool(src_idx3d, x, W1, b1.reshape(1, HID))
    tbl = _pool_proj(sums, W2, b2.reshape(1, HID), W3[2:])
    g = _make_sc_gather()(tbl, tgt_idx3)
    out = _tgt_mlp(g, tgt_coords, W3[:2], b3.reshape(1, HID),
                   W4, b4.reshape(1, OUT_C))
    return out


# speedup vs baseline: 2.5009x; 1.0065x over previous
import functools

import jax
import jax.numpy as jnp
from jax import lax
from jax.experimental import pallas as pl
from jax.experimental.pallas import tpu as pltpu
from jax.experimental.pallas import tpu_sc as plsc

IN_C = 128
OUT_C = 128
HID = 256
NX = 8
NY = 8
NB = 4
NSEG = NB * NX * NY
N = 50000
NPAD = 50176
BLK = 512
NGRID = NPAD // BLK
NC = 2
NS = 16
NW = NC * NS
RPW = NPAD // NW
SB = 112
NBLK = RPW // SB
UW = 384


def _src_pool_body(idx_ref, x_ref, w1_ref, b1_ref, sums_ref):
    i = pl.program_id(0)
    u = jnp.dot(x_ref[...], w1_ref[...], preferred_element_type=jnp.float32)
    u = jax.nn.gelu(u + b1_ref[...])
    u_ext = jnp.concatenate(
        [u, jnp.ones((BLK, UW - HID), jnp.float32)], axis=1)
    rid = i * BLK + jax.lax.broadcasted_iota(jnp.int32, (BLK, 1), 0)
    u_ext = jnp.where(rid < N, u_ext, 0.0)
    seg = idx_ref[0]
    oht = (seg == jax.lax.broadcasted_iota(jnp.int32, (NSEG, BLK), 0)
           ).astype(jnp.float32)
    contrib = jnp.dot(oht, u_ext,
                      preferred_element_type=jnp.float32)

    @pl.when(i == 0)
    def _():
        sums_ref[...] = contrib

    @pl.when(i > 0)
    def _():
        sums_ref[...] += contrib


_src_pool = pl.pallas_call(
    _src_pool_body,
    grid=(NGRID,),
    in_specs=[
        pl.BlockSpec((1, 1, BLK), lambda i: (i, 0, 0)),
        pl.BlockSpec((BLK, IN_C), lambda i: (i, 0)),
        pl.BlockSpec((IN_C, HID), lambda i: (0, 0)),
        pl.BlockSpec((1, HID), lambda i: (0, 0)),
    ],
    out_specs=pl.BlockSpec((NSEG, UW), lambda i: (0, 0)),
    out_shape=jax.ShapeDtypeStruct((NSEG, UW), jnp.float32),
    compiler_params=pltpu.CompilerParams(dimension_semantics=("arbitrary",)),
)


def _pool_proj_body(sums_ref, w2_ref, b2_ref, w3p_ref, tbl1_ref, tbl2_ref):
    full = sums_ref[...]
    su = full[:, 0:HID]
    cnt0 = full[:, HID:HID + 1]
    m = jnp.dot(su, w2_ref[...], preferred_element_type=jnp.float32)
    m = m + cnt0 * b2_ref[...]
    pooled = m / jnp.maximum(cnt0, 1.0)
    tbl = jnp.dot(pooled, w3p_ref[...], preferred_element_type=jnp.float32)
    tbl1_ref[...] = tbl[:, 0:HID // 2]
    tbl2_ref[...] = tbl[:, HID // 2:HID]


NREP = NW

_pool_proj = pl.pallas_call(
    _pool_proj_body,
    grid=(NREP,),
    in_specs=[
        pl.BlockSpec((NSEG, UW), lambda r: (0, 0)),
        pl.BlockSpec((HID, HID), lambda r: (0, 0)),
        pl.BlockSpec((1, HID), lambda r: (0, 0)),
        pl.BlockSpec((HID, HID), lambda r: (0, 0)),
    ],
    out_specs=[
        pl.BlockSpec((NSEG, HID // 2), lambda r: (r, 0)),
        pl.BlockSpec((NSEG, HID // 2), lambda r: (r, 0)),
    ],
    out_shape=[
        jax.ShapeDtypeStruct((NREP * NSEG, HID // 2), jnp.float32),
        jax.ShapeDtypeStruct((NREP * NSEG, HID // 2), jnp.float32),
    ],
    compiler_params=pltpu.CompilerParams(dimension_semantics=("parallel",)),
)


@functools.lru_cache(maxsize=None)
def _make_sc_gather():
    mesh = plsc.VectorSubcoreMesh(core_axis_name="c", subcore_axis_name="s")

    @functools.partial(
        pl.kernel,
        mesh=mesh,
        out_type=[
            jax.ShapeDtypeStruct((NPAD, HID // 2), jnp.float32),
            jax.ShapeDtypeStruct((NPAD, HID // 2), jnp.float32),
        ],
        scratch_types=[
            pltpu.VMEM((NBLK, SB), jnp.int32),
            pltpu.VMEM((SB, HID // 2), jnp.float32),
            pltpu.VMEM((SB, HID // 2), jnp.float32),
            pltpu.SemaphoreType.DMA,
            pltpu.SemaphoreType.DMA,
        ],
    )
    def _sc_gather(tbl1_hbm, tbl2_hbm, idx_hbm, g1_out, g2_out,
                   idx_v, rows1_v, rows2_v, sem1, sem2):
        c = lax.axis_index("c")
        s = lax.axis_index("s")
        w = s * NC + c
        pltpu.sync_copy(idx_hbm.at[w], idx_v)
        base = w * RPW
        for j in range(NBLK):
            a1 = pltpu.async_copy(tbl1_hbm.at[idx_v.at[j]], rows1_v, sem1)
            a2 = pltpu.async_copy(tbl2_hbm.at[idx_v.at[j]], rows2_v, sem2)
            a1.wait()
            pltpu.sync_copy(rows1_v, g1_out.at[pl.ds(base + j * SB, SB)])
            a2.wait()
            pltpu.sync_copy(rows2_v, g2_out.at[pl.ds(base + j * SB, SB)])

    return _sc_gather


def _tgt_mlp_body(g1_ref, g2_ref, c_ref, w3c_ref, b3_ref, w4_ref, b4_ref,
                  o_ref):
    cc = c_ref[...]
    w3c = w3c_ref[...]
    HH = HID // 2
    ct = cc[:, 0:1] * w3c[0:1, :] + cc[:, 1:2] * w3c[1:2, :]
    b3 = b3_ref[...]
    h1 = jax.nn.gelu(g1_ref[...] + ct[:, 0:HH] + b3[:, 0:HH])
    h2 = jax.nn.gelu(g2_ref[...] + ct[:, HH:HID] + b3[:, HH:HID])
    w4 = w4_ref[...]
    o_ref[...] = (jnp.dot(h1, w4[0:HH], preferred_element_type=jnp.float32)
                  + jnp.dot(h2, w4[HH:HID],
                            preferred_element_type=jnp.float32)
                  + b4_ref[...])


_tgt_mlp = pl.pallas_call(
    _tgt_mlp_body,
    grid=(NGRID,),
    in_specs=[
        pl.BlockSpec((BLK, HID // 2), lambda i: (i, 0)),
        pl.BlockSpec((BLK, HID // 2), lambda i: (i, 0)),
        pl.BlockSpec((BLK, 2), lambda i: (i, 0)),
        pl.BlockSpec((2, HID), lambda i: (0, 0)),
        pl.BlockSpec((1, HID), lambda i: (0, 0)),
        pl.BlockSpec((HID, OUT_C), lambda i: (0, 0)),
        pl.BlockSpec((1, OUT_C), lambda i: (0, 0)),
    ],
    out_specs=pl.BlockSpec((BLK, OUT_C), lambda i: (i, 0)),
    out_shape=jax.ShapeDtypeStruct((N, OUT_C), jnp.float32),
    compiler_params=pltpu.CompilerParams(dimension_semantics=("parallel",)),
)


def _clusters(coords, batch):
    cx = jnp.clip(jnp.floor(coords[:, 0] * NX).astype(jnp.int32), 0, NX - 1)
    cy = jnp.clip(jnp.floor(coords[:, 1] * NY).astype(jnp.int32), 0, NY - 1)
    return batch.astype(jnp.int32) * (NX * NY) + cx * NY + cy


def kernel(x, src_coords, src_batch, tgt_coords, tgt_batch,
           W1, b1, W2, b2, W3, b3, W4, b4):
    src_sub = _clusters(src_coords, src_batch)
    tgt_sub = _clusters(tgt_coords, tgt_batch)
    src_idx3d = jnp.full((NPAD,), NSEG, jnp.int32).at[:N].set(src_sub)
    src_idx3d = src_idx3d.reshape(NGRID, 1, BLK)
    tgt_idx = jnp.zeros((NPAD,), jnp.int32).at[:N].set(tgt_sub)
    tgt_idx3 = tgt_idx.reshape(NW, NBLK, SB)
    rep_of_w = (jnp.arange(NW, dtype=jnp.int32) % NREP) * NSEG
    tgt_idx3 = tgt_idx3 + rep_of_w[:, None, None]

    sums = _src_pool(src_idx3d, x, W1, b1.reshape(1, HID))
    tbl1, tbl2 = _pool_proj(sums, W2, b2.reshape(1, HID), W3[2:])
    g1, g2 = _make_sc_gather()(tbl1, tbl2, tgt_idx3)
    out = _tgt_mlp(g1, g2, tgt_coords, W3[:2], b3.reshape(1, HID),
                   W4, b4.reshape(1, OUT_C))
    return out


# speedup vs baseline: 2.6228x; 1.0488x over previous
import functools

import jax
import jax.numpy as jnp
from jax import lax
from jax.experimental import pallas as pl
from jax.experimental.pallas import tpu as pltpu
from jax.experimental.pallas import tpu_sc as plsc

IN_C = 128
OUT_C = 128
HID = 256
NX = 8
NY = 8
NB = 4
NSEG = NB * NX * NY
N = 50000
NPAD = 50176
BLK = 512
NGRID = NPAD // BLK
NC = 2
NS = 16
NW = NC * NS
RPW = NPAD // NW
SB = 112
NBLK = RPW // SB
UW = 384


NREP = NW


def _src_pool_body(idx_ref, x_ref, w1_ref, b1_ref, w2_ref, b2_ref, w3p_ref,
                   tbl1_ref, tbl2_ref, sums_ref):
    i = pl.program_id(0)
    u = jnp.dot(x_ref[...], w1_ref[...], preferred_element_type=jnp.float32)
    u = jax.nn.gelu(u + b1_ref[...])
    u_ext = jnp.concatenate(
        [u, jnp.ones((BLK, UW - HID), jnp.float32)], axis=1)
    rid = i * BLK + jax.lax.broadcasted_iota(jnp.int32, (BLK, 1), 0)
    u_ext = jnp.where(rid < N, u_ext, 0.0)
    seg = idx_ref[0]
    oht = (seg == jax.lax.broadcasted_iota(jnp.int32, (NSEG, BLK), 0)
           ).astype(jnp.float32)
    contrib = jnp.dot(oht, u_ext,
                      preferred_element_type=jnp.float32)

    @pl.when(i == 0)
    def _():
        sums_ref[...] = contrib

    @pl.when(i > 0)
    def _():
        sums_ref[...] += contrib

    @pl.when(i == NGRID - 1)
    def _():
        full = sums_ref[...]
        su = full[:, 0:HID]
        cnt0 = full[:, HID:HID + 1]
        m = jnp.dot(su, w2_ref[...], preferred_element_type=jnp.float32)
        m = m + cnt0 * b2_ref[...]
        pooled = m / jnp.maximum(cnt0, 1.0)
        tbl = jnp.dot(pooled, w3p_ref[...],
                      preferred_element_type=jnp.float32)
        HH = HID // 2
        rep1 = jnp.broadcast_to(tbl[None, :, 0:HH], (NREP, NSEG, HH))
        rep2 = jnp.broadcast_to(tbl[None, :, HH:HID], (NREP, NSEG, HH))
        tbl1_ref[...] = rep1.reshape(NREP * NSEG, HH)
        tbl2_ref[...] = rep2.reshape(NREP * NSEG, HH)


_src_pool = pl.pallas_call(
    _src_pool_body,
    grid=(NGRID,),
    in_specs=[
        pl.BlockSpec((1, 1, BLK), lambda i: (i, 0, 0)),
        pl.BlockSpec((BLK, IN_C), lambda i: (i, 0)),
        pl.BlockSpec((IN_C, HID), lambda i: (0, 0)),
        pl.BlockSpec((1, HID), lambda i: (0, 0)),
        pl.BlockSpec((HID, HID), lambda i: (0, 0)),
        pl.BlockSpec((1, HID), lambda i: (0, 0)),
        pl.BlockSpec((HID, HID), lambda i: (0, 0)),
    ],
    out_specs=[
        pl.BlockSpec((NREP * NSEG, HID // 2), lambda i: (0, 0)),
        pl.BlockSpec((NREP * NSEG, HID // 2), lambda i: (0, 0)),
        pl.BlockSpec((NSEG, UW), lambda i: (0, 0)),
    ],
    out_shape=[
        jax.ShapeDtypeStruct((NREP * NSEG, HID // 2), jnp.float32),
        jax.ShapeDtypeStruct((NREP * NSEG, HID // 2), jnp.float32),
        jax.ShapeDtypeStruct((NSEG, UW), jnp.float32),
    ],
    compiler_params=pltpu.CompilerParams(dimension_semantics=("arbitrary",)),
)


@functools.lru_cache(maxsize=None)
def _make_sc_gather():
    mesh = plsc.VectorSubcoreMesh(core_axis_name="c", subcore_axis_name="s")

    @functools.partial(
        pl.kernel,
        mesh=mesh,
        out_type=[
            jax.ShapeDtypeStruct((NPAD, HID // 2), jnp.float32),
            jax.ShapeDtypeStruct((NPAD, HID // 2), jnp.float32),
        ],
        scratch_types=[
            pltpu.VMEM((NBLK, SB), jnp.int32),
            pltpu.VMEM((SB, HID // 2), jnp.float32),
            pltpu.VMEM((SB, HID // 2), jnp.float32),
            pltpu.SemaphoreType.DMA,
            pltpu.SemaphoreType.DMA,
        ],
    )
    def _sc_gather(tbl1_hbm, tbl2_hbm, idx_hbm, g1_out, g2_out,
                   idx_v, rows1_v, rows2_v, sem1, sem2):
        c = lax.axis_index("c")
        s = lax.axis_index("s")
        w = s * NC + c
        pltpu.sync_copy(idx_hbm.at[w], idx_v)
        base = w * RPW
        for j in range(NBLK):
            a1 = pltpu.async_copy(tbl1_hbm.at[idx_v.at[j]], rows1_v, sem1)
            a2 = pltpu.async_copy(tbl2_hbm.at[idx_v.at[j]], rows2_v, sem2)
            a1.wait()
            pltpu.sync_copy(rows1_v, g1_out.at[pl.ds(base + j * SB, SB)])
            a2.wait()
            pltpu.sync_copy(rows2_v, g2_out.at[pl.ds(base + j * SB, SB)])

    return _sc_gather


def _tgt_mlp_body(g1_ref, g2_ref, c_ref, w3c_ref, b3_ref, w4_ref, b4_ref,
                  o_ref):
    cc = c_ref[...]
    w3c = w3c_ref[...]
    HH = HID // 2
    ct = cc[:, 0:1] * w3c[0:1, :] + cc[:, 1:2] * w3c[1:2, :]
    b3 = b3_ref[...]
    h1 = jax.nn.gelu(g1_ref[...] + ct[:, 0:HH] + b3[:, 0:HH])
    h2 = jax.nn.gelu(g2_ref[...] + ct[:, HH:HID] + b3[:, HH:HID])
    w4 = w4_ref[...]
    o_ref[...] = (jnp.dot(h1, w4[0:HH], preferred_element_type=jnp.float32)
                  + jnp.dot(h2, w4[HH:HID],
                            preferred_element_type=jnp.float32)
                  + b4_ref[...])


_tgt_mlp = pl.pallas_call(
    _tgt_mlp_body,
    grid=(NGRID,),
    in_specs=[
        pl.BlockSpec((BLK, HID // 2), lambda i: (i, 0)),
        pl.BlockSpec((BLK, HID // 2), lambda i: (i, 0)),
        pl.BlockSpec((BLK, 2), lambda i: (i, 0)),
        pl.BlockSpec((2, HID), lambda i: (0, 0)),
        pl.BlockSpec((1, HID), lambda i: (0, 0)),
        pl.BlockSpec((HID, OUT_C), lambda i: (0, 0)),
        pl.BlockSpec((1, OUT_C), lambda i: (0, 0)),
    ],
    out_specs=pl.BlockSpec((BLK, OUT_C), lambda i: (i, 0)),
    out_shape=jax.ShapeDtypeStruct((N, OUT_C), jnp.float32),
    compiler_params=pltpu.CompilerParams(dimension_semantics=("parallel",)),
)


def _clusters(coords, batch):
    cx = jnp.clip(jnp.floor(coords[:, 0] * NX).astype(jnp.int32), 0, NX - 1)
    cy = jnp.clip(jnp.floor(coords[:, 1] * NY).astype(jnp.int32), 0, NY - 1)
    return batch.astype(jnp.int32) * (NX * NY) + cx * NY + cy


def kernel(x, src_coords, src_batch, tgt_coords, tgt_batch,
           W1, b1, W2, b2, W3, b3, W4, b4):
    src_sub = _clusters(src_coords, src_batch)
    tgt_sub = _clusters(tgt_coords, tgt_batch)
    src_idx3d = jnp.full((NPAD,), NSEG, jnp.int32).at[:N].set(src_sub)
    src_idx3d = src_idx3d.reshape(NGRID, 1, BLK)
    tgt_idx = jnp.zeros((NPAD,), jnp.int32).at[:N].set(tgt_sub)
    tgt_idx3 = tgt_idx.reshape(NW, NBLK, SB)
    rep_of_w = (jnp.arange(NW, dtype=jnp.int32) % NREP) * NSEG
    tgt_idx3 = tgt_idx3 + rep_of_w[:, None, None]

    tbl1, tbl2, _ = _src_pool(src_idx3d, x, W1, b1.reshape(1, HID),
                              W2, b2.reshape(1, HID), W3[2:])
    g1, g2 = _make_sc_gather()(tbl1, tbl2, tgt_idx3)
    out = _tgt_mlp(g1, g2, tgt_coords, W3[:2], b3.reshape(1, HID),
                   W4, b4.reshape(1, OUT_C))
    return out


# speedup vs baseline: 2.9544x; 1.1264x over previous
import functools

import jax
import jax.numpy as jnp
from jax import lax
from jax.experimental import pallas as pl
from jax.experimental.pallas import tpu as pltpu
from jax.experimental.pallas import tpu_sc as plsc

IN_C = 128
OUT_C = 128
HID = 256
NX = 8
NY = 8
NB = 4
NSEG = NB * NX * NY
N = 50000
NPAD = 50176
BLK = 512
NGRID = NPAD // BLK
NC = 2
NS = 16
NW = NC * NS
RPW = NPAD // NW
SB = 112
NBLK = RPW // SB
UW = 384


NREP = NW


def _src_pool_body(idx_ref, x_ref, w1_ref, b1_ref, w2_ref, b2_ref, w3p_ref,
                   tblp_ref, sums_ref):
    i = pl.program_id(0)
    u = jnp.dot(x_ref[...], w1_ref[...], preferred_element_type=jnp.float32)
    u = jax.nn.gelu(u + b1_ref[...])
    u_ext = jnp.concatenate(
        [u, jnp.ones((BLK, UW - HID), jnp.float32)], axis=1)
    rid = i * BLK + jax.lax.broadcasted_iota(jnp.int32, (BLK, 1), 0)
    u_ext = jnp.where(rid < N, u_ext, 0.0)
    seg = idx_ref[0]
    oht = (seg == jax.lax.broadcasted_iota(jnp.int32, (NSEG, BLK), 0)
           ).astype(jnp.float32)
    contrib = jnp.dot(oht, u_ext,
                      preferred_element_type=jnp.float32)

    @pl.when(i == 0)
    def _():
        sums_ref[...] = contrib

    @pl.when(i > 0)
    def _():
        sums_ref[...] += contrib

    @pl.when(i == NGRID - 1)
    def _():
        full = sums_ref[...]
        su = full[:, 0:HID]
        cnt0 = full[:, HID:HID + 1]
        m = jnp.dot(su, w2_ref[...], preferred_element_type=jnp.float32)
        m = m + cnt0 * b2_ref[...]
        pooled = m / jnp.maximum(cnt0, 1.0)
        tbl = jnp.dot(pooled, w3p_ref[...],
                      preferred_element_type=jnp.float32)
        HH = HID // 2
        bits = jax.lax.bitcast_convert_type(tbl, jnp.uint32)
        half = jnp.uint32(0x8000)
        hi = (bits[:, 0:HH] + half) & jnp.uint32(0xFFFF0000)
        lo = (bits[:, HH:HID] + half) >> 16
        packed = jax.lax.bitcast_convert_type(hi | lo, jnp.int32)
        rep = jnp.broadcast_to(packed[None], (NREP, NSEG, HH))
        tblp_ref[...] = rep.reshape(NREP * NSEG, HH)


_src_pool = pl.pallas_call(
    _src_pool_body,
    grid=(NGRID,),
    in_specs=[
        pl.BlockSpec((1, 1, BLK), lambda i: (i, 0, 0)),
        pl.BlockSpec((BLK, IN_C), lambda i: (i, 0)),
        pl.BlockSpec((IN_C, HID), lambda i: (0, 0)),
        pl.BlockSpec((1, HID), lambda i: (0, 0)),
        pl.BlockSpec((HID, HID), lambda i: (0, 0)),
        pl.BlockSpec((1, HID), lambda i: (0, 0)),
        pl.BlockSpec((HID, HID), lambda i: (0, 0)),
    ],
    out_specs=[
        pl.BlockSpec((NREP * NSEG, HID // 2), lambda i: (0, 0)),
        pl.BlockSpec((NSEG, UW), lambda i: (0, 0)),
    ],
    out_shape=[
        jax.ShapeDtypeStruct((NREP * NSEG, HID // 2), jnp.int32),
        jax.ShapeDtypeStruct((NSEG, UW), jnp.float32),
    ],
    compiler_params=pltpu.CompilerParams(dimension_semantics=("arbitrary",)),
)


@functools.lru_cache(maxsize=None)
def _make_sc_gather():
    mesh = plsc.VectorSubcoreMesh(core_axis_name="c", subcore_axis_name="s")

    @functools.partial(
        pl.kernel,
        mesh=mesh,
        out_type=[
            jax.ShapeDtypeStruct((NPAD, HID // 2), jnp.int32),
        ],
        scratch_types=[
            pltpu.VMEM((NBLK, SB), jnp.int32),
            pltpu.VMEM((SB, HID // 2), jnp.int32),
            pltpu.VMEM((SB, HID // 2), jnp.int32),
            pltpu.SemaphoreType.DMA,
            pltpu.SemaphoreType.DMA,
        ],
    )
    def _sc_gather(tbl_hbm, idx_hbm, g_out,
                   idx_v, rows_a, rows_b, sem_a, sem_b):
        c = lax.axis_index("c")
        s = lax.axis_index("s")
        w = s * NC + c
        pltpu.sync_copy(idx_hbm.at[w], idx_v)
        base = w * RPW
        bufs = (rows_a, rows_b)
        sems = (sem_a, sem_b)
        pend = pltpu.async_copy(tbl_hbm.at[idx_v.at[0]], rows_a, sem_a)
        for j in range(NBLK):
            if j + 1 < NBLK:
                nxt = pltpu.async_copy(tbl_hbm.at[idx_v.at[j + 1]],
                                       bufs[(j + 1) % 2], sems[(j + 1) % 2])
            pend.wait()
            pltpu.sync_copy(bufs[j % 2],
                            g_out.at[pl.ds(base + j * SB, SB)])
            if j + 1 < NBLK:
                pend = nxt

    return _sc_gather


def _tgt_mlp_body(g_ref, c_ref, w3c_ref, b3_ref, w4_ref, b4_ref,
                  o_ref):
    cc = c_ref[...]
    w3c = w3c_ref[...]
    HH = HID // 2
    ct = cc[:, 0:1] * w3c[0:1, :] + cc[:, 1:2] * w3c[1:2, :]
    b3 = b3_ref[...]
    bits = jax.lax.bitcast_convert_type(g_ref[...], jnp.uint32)
    g1 = jax.lax.bitcast_convert_type(bits & jnp.uint32(0xFFFF0000),
                                      jnp.float32)
    g2 = jax.lax.bitcast_convert_type(bits << 16, jnp.float32)
    h1 = jax.nn.gelu(g1 + ct[:, 0:HH] + b3[:, 0:HH])
    h2 = jax.nn.gelu(g2 + ct[:, HH:HID] + b3[:, HH:HID])
    w4 = w4_ref[...]
    o_ref[...] = (jnp.dot(h1, w4[0:HH], preferred_element_type=jnp.float32)
                  + jnp.dot(h2, w4[HH:HID],
                            preferred_element_type=jnp.float32)
                  + b4_ref[...])


_tgt_mlp = pl.pallas_call(
    _tgt_mlp_body,
    grid=(NGRID,),
    in_specs=[
        pl.BlockSpec((BLK, HID // 2), lambda i: (i, 0)),
        pl.BlockSpec((BLK, 2), lambda i: (i, 0)),
        pl.BlockSpec((2, HID), lambda i: (0, 0)),
        pl.BlockSpec((1, HID), lambda i: (0, 0)),
        pl.BlockSpec((HID, OUT_C), lambda i: (0, 0)),
        pl.BlockSpec((1, OUT_C), lambda i: (0, 0)),
    ],
    out_specs=pl.BlockSpec((BLK, OUT_C), lambda i: (i, 0)),
    out_shape=jax.ShapeDtypeStruct((N, OUT_C), jnp.float32),
    compiler_params=pltpu.CompilerParams(dimension_semantics=("parallel",)),
)


def _clusters(coords, batch):
    cx = jnp.clip(jnp.floor(coords[:, 0] * NX).astype(jnp.int32), 0, NX - 1)
    cy = jnp.clip(jnp.floor(coords[:, 1] * NY).astype(jnp.int32), 0, NY - 1)
    return batch.astype(jnp.int32) * (NX * NY) + cx * NY + cy


def kernel(x, src_coords, src_batch, tgt_coords, tgt_batch,
           W1, b1, W2, b2, W3, b3, W4, b4):
    src_sub = _clusters(src_coords, src_batch)
    tgt_sub = _clusters(tgt_coords, tgt_batch)
    src_idx3d = jnp.full((NPAD,), NSEG, jnp.int32).at[:N].set(src_sub)
    src_idx3d = src_idx3d.reshape(NGRID, 1, BLK)
    tgt_idx = jnp.zeros((NPAD,), jnp.int32).at[:N].set(tgt_sub)
    tgt_idx3 = tgt_idx.reshape(NW, NBLK, SB)
    rep_of_w = (jnp.arange(NW, dtype=jnp.int32) % NREP) * NSEG
    tgt_idx3 = tgt_idx3 + rep_of_w[:, None, None]

    tblp, _ = _src_pool(src_idx3d, x, W1, b1.reshape(1, HID),
                        W2, b2.reshape(1, HID), W3[2:])
    (g,) = _make_sc_gather()(tblp, tgt_idx3)
    out = _tgt_mlp(g, tgt_coords, W3[:2], b3.reshape(1, HID),
                   W4, b4.reshape(1, OUT_C))
    return out
